# baseline (speedup 1.0000x reference)
import functools

import jax
import jax.numpy as jnp
from jax.experimental import pallas as pl
from jax.experimental.pallas import tpu as pltpu

_LANE = 128
_BM = 2048
_BN = 512
_NCHUNK = 16


def _ceil_to(v: int, m: int) -> int:
    return ((v + m - 1) // m) * m


def _pick_block(dim: int, cap: int) -> int:
    b = min(cap, dim)
    while dim % b:
        b -= _LANE
    return b


def _mm_kernel(x_ref, w_ref, o_ref):
    o_ref[...] = jnp.dot(x_ref[...], w_ref[...],
                         preferred_element_type=jnp.float32)


def _mm_manual_kernel(x_hbm, w_ref, o_ref, x_vmem, sems, *, nk: int):
    i = pl.program_id(0)
    bm, kdim = x_vmem.shape
    ck = kdim // nk

    def chunk_copy(k):
        return pltpu.make_async_copy(
            x_hbm.at[pl.ds(i * bm, bm), pl.ds(k * ck, ck)],
            x_vmem.at[:, pl.ds(k * ck, ck)],
            sems.at[k])

    @pl.when(pl.program_id(1) == 0)
    def _first_col():
        for k in range(nk):
            chunk_copy(k).start()
        for k in range(nk):
            chunk_copy(k).wait()
            part = jnp.dot(x_vmem[:, k * ck:(k + 1) * ck],
                           w_ref[k * ck:(k + 1) * ck, :],
                           preferred_element_type=jnp.float32)
            if k == 0:
                o_ref[...] = part
            else:
                o_ref[...] += part

    @pl.when(pl.program_id(1) != 0)
    def _rest():
        o_ref[...] = jnp.dot(x_vmem[...], w_ref[...],
                             preferred_element_type=jnp.float32)


def kernel(x, weight):
    M, K = x.shape
    K2, N = weight.shape
    assert K == K2, "inner dims must match"

    M_pad, K_pad, N_pad = (_ceil_to(M, _LANE), _ceil_to(K, _LANE),
                           _ceil_to(N, _LANE))
    xp, wp = x, weight
    if (M_pad, K_pad) != (M, K):
        xp = jnp.pad(xp, ((0, M_pad - M), (0, K_pad - K)))
    if (K_pad, N_pad) != (K, N):
        wp = jnp.pad(wp, ((0, K_pad - K), (0, N_pad - N)))

    manual = (M_pad % _BM == 0 and N_pad % _BN == 0
              and K_pad % (_NCHUNK * _LANE) == 0)

    if manual:
        bm, bn, nk = _BM, _BN, _NCHUNK
        grid = (M_pad // bm, N_pad // bn)
        out = pl.pallas_call(
            functools.partial(_mm_manual_kernel, nk=nk),
            out_shape=jax.ShapeDtypeStruct((M_pad, N_pad), jnp.float32),
            grid=grid,
            in_specs=[
                pl.BlockSpec(memory_space=pl.ANY),
                pl.BlockSpec((K_pad, bn), lambda i, j: (0, j)),
            ],
            out_specs=pl.BlockSpec((bm, bn), lambda i, j: (i, j)),
            scratch_shapes=[
                pltpu.VMEM((bm, K_pad), jnp.float32),
                pltpu.SemaphoreType.DMA((nk,)),
            ],
            compiler_params=pltpu.CompilerParams(
                dimension_semantics=("parallel", "arbitrary"),
                vmem_limit_bytes=64 * 1024 * 1024,
            ),
        )(xp, wp)
    else:
        bm = _pick_block(M_pad, _BM)
        bn = _pick_block(N_pad, _BN)
        grid = (M_pad // bm, N_pad // bn)
        out = pl.pallas_call(
            _mm_kernel,
            out_shape=jax.ShapeDtypeStruct((M_pad, N_pad), jnp.float32),
            grid=grid,
            in_specs=[
                pl.BlockSpec((bm, K_pad), lambda i, j: (i, 0),
                             pipeline_mode=pl.Buffered(buffer_count=1)),
                pl.BlockSpec((K_pad, bn), lambda i, j: (0, j)),
            ],
            out_specs=pl.BlockSpec((bm, bn), lambda i, j: (i, j)),
            compiler_params=pltpu.CompilerParams(
                dimension_semantics=("parallel", "parallel"),
                vmem_limit_bytes=64 * 1024 * 1024,
            ),
        )(xp, wp)

    if (M_pad, N_pad) != (M, N):
        out = out[:M, :N]
    return out


# speedup vs baseline: 1.0090x; 1.0090x over previous
import functools

import jax
import jax.numpy as jnp
from jax.experimental import pallas as pl
from jax.experimental.pallas import tpu as pltpu

_LANE = 128
_BM = 2048
_BN = 512
_NCHUNK = 8


def _ceil_to(v: int, m: int) -> int:
    return ((v + m - 1) // m) * m


def _pick_block(dim: int, cap: int) -> int:
    b = min(cap, dim)
    while dim % b:
        b -= _LANE
    return b


def _mm_kernel(x_ref, w_ref, o_ref):
    o_ref[...] = jnp.dot(x_ref[...], w_ref[...],
                         preferred_element_type=jnp.float32)


def _mm_manual_kernel(x_hbm, w_ref, o_ref, x_vmem, sems, *, nk: int):
    i = pl.program_id(0)
    bm, kdim = x_vmem.shape
    ck = kdim // nk

    def chunk_copy(k):
        return pltpu.make_async_copy(
            x_hbm.at[pl.ds(i * bm, bm), pl.ds(k * ck, ck)],
            x_vmem.at[:, pl.ds(k * ck, ck)],
            sems.at[k])

    @pl.when(pl.program_id(1) == 0)
    def _first_col():
        for k in range(nk):
            chunk_copy(k).start()
        for k in range(nk):
            chunk_copy(k).wait()
            part = jnp.dot(x_vmem[:, k * ck:(k + 1) * ck],
                           w_ref[k * ck:(k + 1) * ck, :],
                           preferred_element_type=jnp.float32)
            if k == 0:
                o_ref[...] = part
            else:
                o_ref[...] += part

    @pl.when(pl.program_id(1) != 0)
    def _rest():
        o_ref[...] = jnp.dot(x_vmem[...], w_ref[...],
                             preferred_element_type=jnp.float32)


def kernel(x, weight):
    M, K = x.shape
    K2, N = weight.shape
    assert K == K2, "inner dims must match"

    M_pad, K_pad, N_pad = (_ceil_to(M, _LANE), _ceil_to(K, _LANE),
                           _ceil_to(N, _LANE))
    xp, wp = x, weight
    if (M_pad, K_pad) != (M, K):
        xp = jnp.pad(xp, ((0, M_pad - M), (0, K_pad - K)))
    if (K_pad, N_pad) != (K, N):
        wp = jnp.pad(wp, ((0, K_pad - K), (0, N_pad - N)))

    manual = (M_pad % _BM == 0 and N_pad % _BN == 0
              and K_pad % (_NCHUNK * _LANE) == 0)

    if manual:
        bm, bn, nk = _BM, _BN, _NCHUNK
        grid = (M_pad // bm, N_pad // bn)
        out = pl.pallas_call(
            functools.partial(_mm_manual_kernel, nk=nk),
            out_shape=jax.ShapeDtypeStruct((M_pad, N_pad), jnp.float32),
            grid=grid,
            in_specs=[
                pl.BlockSpec(memory_space=pl.ANY),
                pl.BlockSpec((K_pad, bn), lambda i, j: (0, j)),
            ],
            out_specs=pl.BlockSpec((bm, bn), lambda i, j: (i, j)),
            scratch_shapes=[
                pltpu.VMEM((bm, K_pad), jnp.float32),
                pltpu.SemaphoreType.DMA((nk,)),
            ],
            compiler_params=pltpu.CompilerParams(
                dimension_semantics=("parallel", "arbitrary"),
                vmem_limit_bytes=64 * 1024 * 1024,
            ),
        )(xp, wp)
    else:
        bm = _pick_block(M_pad, _BM)
        bn = _pick_block(N_pad, _BN)
        grid = (M_pad // bm, N_pad // bn)
        out = pl.pallas_call(
            _mm_kernel,
            out_shape=jax.ShapeDtypeStruct((M_pad, N_pad), jnp.float32),
            grid=grid,
            in_specs=[
                pl.BlockSpec((bm, K_pad), lambda i, j: (i, 0),
                             pipeline_mode=pl.Buffered(buffer_count=1)),
                pl.BlockSpec((K_pad, bn), lambda i, j: (0, j)),
            ],
            out_specs=pl.BlockSpec((bm, bn), lambda i, j: (i, j)),
            compiler_params=pltpu.CompilerParams(
                dimension_semantics=("parallel", "parallel"),
                vmem_limit_bytes=64 * 1024 * 1024,
            ),
        )(xp, wp)

    if (M_pad, N_pad) != (M, N):
        out = out[:M, :N]
    return out
